# baseline (speedup 1.0000x reference)
import jax
import jax.numpy as jnp
from jax.experimental import pallas as pl
import jax.experimental.pallas.tpu as pltpu

ROWS = 128
VOCAB = 100000
RB = 16
NSTREAM = 4
GRID = ROWS // (RB * NSTREAM)
BAND = ROWS // NSTREAM
W = 1024
NCHUNK = VOCAB // W
REM = VOCAB - NCHUNK * W


def _argmax_rows(x_ref, out_ref):
    def body(k, carry):
        vmax, vchunk = carry
        chunk = x_ref[:, pl.ds(k * W, W)]
        better = chunk > vmax
        vchunk = jnp.where(better, k, vchunk)
        vmax = jnp.where(better, chunk, vmax)
        return vmax, vchunk

    vmax = x_ref[:, :W]
    vchunk = jnp.zeros((RB, W), jnp.int32)
    vmax, vchunk = jax.lax.fori_loop(1, NCHUNK, body, (vmax, vchunk))

    col = jax.lax.broadcasted_iota(jnp.int32, (RB, W), 1)
    chunk = x_ref[:, NCHUNK * W:(NCHUNK + 1) * W]
    chunk = jnp.where(col < REM, chunk, -jnp.inf)
    better = chunk > vmax
    vchunk = jnp.where(better, NCHUNK, vchunk)
    vmax = jnp.where(better, chunk, vmax)

    m = jnp.max(vmax, axis=1, keepdims=True)
    gcol = vchunk * W + col
    cand = jnp.where(vmax == m, gcol, jnp.int32(2**31 - 1))
    out_ref[...] = jnp.min(cand, axis=1, keepdims=True)


def _argmax_body(*refs):
    x_refs = refs[:NSTREAM]
    out_refs = refs[NSTREAM:]
    for x_ref, out_ref in zip(x_refs, out_refs):
        _argmax_rows(x_ref, out_ref)


@jax.jit
def _argmax_pallas(m_logits):
    bands = [m_logits[j * BAND:(j + 1) * BAND] for j in range(NSTREAM)]
    outs = pl.pallas_call(
        _argmax_body,
        grid=(GRID,),
        in_specs=[pl.BlockSpec((RB, (NCHUNK + 1) * W), lambda i: (i, 0))
                  for _ in range(NSTREAM)],
        out_specs=[pl.BlockSpec((RB, 1), lambda i: (i, 0))
                   for _ in range(NSTREAM)],
        out_shape=[jax.ShapeDtypeStruct((BAND, 1), jnp.int32)
                   for _ in range(NSTREAM)],
    )(*bands)
    return jnp.concatenate(outs, axis=0)


def kernel(m_logits):
    token = _argmax_pallas(m_logits.astype(jnp.float32))
    return token.astype(jnp.int64)


# speedup vs baseline: 3.7958x; 3.7958x over previous
import jax
import jax.numpy as jnp
from jax.experimental import pallas as pl
import jax.experimental.pallas.tpu as pltpu

ROWS = 128
VOCAB = 100000
RB = 16
NSTREAM = 4
GRID = ROWS // (RB * NSTREAM)
BAND = ROWS // NSTREAM
W = 1024
NCHUNK = VOCAB // W
REM = VOCAB - NCHUNK * W


def _argmax_rows(x_ref, out_ref):
    def body(k, carry):
        vmax, vchunk = carry
        chunk = x_ref[:, pl.ds(k * W, W)]
        better = chunk > vmax
        vchunk = jnp.where(better, k, vchunk)
        vmax = jnp.where(better, chunk, vmax)
        return vmax, vchunk

    vmax = x_ref[:, :W]
    vchunk = jnp.zeros((RB, W), jnp.int32)
    vmax, vchunk = jax.lax.fori_loop(1, NCHUNK, body, (vmax, vchunk))

    col = jax.lax.broadcasted_iota(jnp.int32, (RB, W), 1)
    chunk = x_ref[:, NCHUNK * W:(NCHUNK + 1) * W]
    chunk = jnp.where(col < REM, chunk, -jnp.inf)
    better = chunk > vmax
    vchunk = jnp.where(better, NCHUNK, vchunk)
    vmax = jnp.where(better, chunk, vmax)

    m = jnp.max(vmax, axis=1, keepdims=True)
    gcol = vchunk * W + col
    cand = jnp.where(vmax == m, gcol, jnp.int32(2**31 - 1))
    out_ref[...] = jnp.min(cand, axis=1, keepdims=True)


def _argmax_body(*refs):
    x_refs = refs[:NSTREAM]
    out_refs = refs[NSTREAM:]
    for x_ref, out_ref in zip(x_refs, out_refs):
        _argmax_rows(x_ref, out_ref)


@jax.jit
def _argmax_pallas(m_logits):
    outs = pl.pallas_call(
        _argmax_body,
        grid=(GRID,),
        in_specs=[pl.BlockSpec((RB, (NCHUNK + 1) * W),
                               lambda i, j=j: (j * GRID + i, 0))
                  for j in range(NSTREAM)],
        out_specs=[pl.BlockSpec((RB, 1), lambda i: (i, 0))
                   for _ in range(NSTREAM)],
        out_shape=[jax.ShapeDtypeStruct((BAND, 1), jnp.int32)
                   for _ in range(NSTREAM)],
    )(*([m_logits] * NSTREAM))
    return jnp.concatenate(outs, axis=0)


def kernel(m_logits):
    token = _argmax_pallas(m_logits.astype(jnp.float32))
    return token.astype(jnp.int64)
